# baseline (speedup 1.0000x reference)
import functools

import jax
import jax.numpy as jnp
from jax.experimental import pallas as pl
from jax.experimental.pallas import tpu as pltpu

_B, _C, _N, _K = 4, 3, 4096, 32
_R = 256

_NEG = -3.0e38


def _leaky(v):
    return jnp.maximum(v, 0.2 * v)


def _knn_conv_body(xf_ref, xr_ref, w1a_ref, wb_ref, w2_ref, o_ref):
    xf = xf_ref[0]
    xr = xr_ref[0]

    xxf = jnp.sum(xf * xf, axis=0)
    xxr = jnp.sum(xr * xr, axis=0)
    g = jax.lax.dot_general(
        xr, xf, (((0,), (0,)), ((), ())),
        preferred_element_type=jnp.float32)
    d = 2.0 * g - xxr[:, None] - xxf[None, :]

    base = jax.lax.dot_general(
        xr, wb_ref[...], (((0,), (0,)), ((), ())),
        preferred_element_type=jnp.float32)
    w1a = w1a_ref[...]
    w2 = w2_ref[...]

    ii = jax.lax.broadcasted_iota(jnp.int32, d.shape, 1)
    r0 = pl.program_id(1)
    gi = r0 * d.shape[0] + jax.lax.broadcasted_iota(jnp.int32, (d.shape[0], 1), 0)
    d = jnp.where(ii == gi, _NEG, d)
    h1s = _leaky(jax.lax.dot_general(
        xr, w1a, (((0,), (0,)), ((), ())),
        preferred_element_type=jnp.float32) + base)
    acc = _leaky(jax.lax.dot_general(
        h1s, w2, (((1,), (0,)), ((), ())),
        preferred_element_type=jnp.float32))
    for _ in range(_K - 1):
        am = jnp.argmax(d, axis=1)[:, None]
        hit = ii == am
        oh = jnp.where(hit, 1.0, 0.0)
        nb = jax.lax.dot_general(
            oh, xf, (((1,), (1,)), ((), ())),
            preferred_element_type=jnp.float32)
        d = jnp.where(hit, _NEG, d)
        h1 = _leaky(jax.lax.dot_general(
            nb, w1a, (((1,), (0,)), ((), ())),
            preferred_element_type=jnp.float32) + base)
        h2 = _leaky(jax.lax.dot_general(
            h1, w2, (((1,), (0,)), ((), ())),
            preferred_element_type=jnp.float32))
        acc = jnp.maximum(acc, h2)
    o_ref[0] = acc


@jax.jit
def kernel(x, W1, W2):
    b, c, n = x.shape
    xp = jnp.pad(x, ((0, 0), (0, 8 - c), (0, 0)))
    w1a = jnp.pad(W1[:, :c].T, ((0, 8 - c), (0, 0)))
    wb = jnp.pad((W1[:, c:] - W1[:, :c]).T, ((0, 8 - c), (0, 0)))
    w2t = W2.T

    out = pl.pallas_call(
        _knn_conv_body,
        grid=(b, n // _R),
        in_specs=[
            pl.BlockSpec((1, 8, n), lambda i, j: (i, 0, 0)),
            pl.BlockSpec((1, 8, _R), lambda i, j: (i, 0, j)),
            pl.BlockSpec((8, 64), lambda i, j: (0, 0)),
            pl.BlockSpec((8, 64), lambda i, j: (0, 0)),
            pl.BlockSpec((64, 64), lambda i, j: (0, 0)),
        ],
        out_specs=pl.BlockSpec((1, _R, 64), lambda i, j: (i, j, 0)),
        out_shape=jax.ShapeDtypeStruct((b, n, 64), jnp.float32),
    )(xp, xp, w1a, wb, w2t)
    return jnp.swapaxes(out, 1, 2)


# speedup vs baseline: 2.3223x; 2.3223x over previous
import functools

import jax
import jax.numpy as jnp
from jax import lax
from jax.experimental import pallas as pl
from jax.experimental.pallas import tpu as pltpu
from jax.experimental.pallas import tpu_sc as plsc

_B, _C, _N, _K = 4, 3, 4096, 32
_R = 256

_NEG = -3.0e38

_NW = 32


def _leaky(v):
    return jnp.maximum(v, 0.2 * v)


def _topk_body(xf_ref, xr_ref, idx_ref):
    xf = xf_ref[0]
    xr = xr_ref[0]

    xxf = jnp.sum(xf * xf, axis=0)
    xxr = jnp.sum(xr * xr, axis=0)
    g = lax.dot_general(
        xr, xf, (((0,), (0,)), ((), ())),
        preferred_element_type=jnp.float32)
    d = 2.0 * g - xxr[:, None] - xxf[None, :]

    ii = lax.broadcasted_iota(jnp.int32, d.shape, 1)
    r0 = pl.program_id(1)
    gi = r0 * d.shape[0] + lax.broadcasted_iota(jnp.int32, (d.shape[0], 1), 0)
    d = jnp.where(ii == gi, _NEG, d)

    boff = pl.program_id(0) * xf.shape[1]
    cols = [gi]
    for _ in range(_K - 1):
        am = jnp.argmax(d, axis=1)[:, None]
        cols.append(am)
        d = jnp.where(ii == am, _NEG, d)
    idx_ref[0] = jnp.concatenate(cols, axis=1) + boff


def _sc_gather(t0, t1, t2, idxf):
    total = idxf.shape[0]
    bn = t0.shape[0]
    per_w = total // _NW
    chunk = 2048
    mesh = plsc.VectorSubcoreMesh(core_axis_name="c", subcore_axis_name="s")

    @functools.partial(
        pl.kernel, mesh=mesh,
        compiler_params=pltpu.CompilerParams(needs_layout_passes=False),
        out_type=jax.ShapeDtypeStruct((3, 1, total), jnp.float32),
        scratch_types=[
            pltpu.VMEM((bn,), jnp.float32),
            pltpu.VMEM((bn,), jnp.float32),
            pltpu.VMEM((bn,), jnp.float32),
            pltpu.VMEM((per_w,), jnp.int32),
            pltpu.VMEM((chunk,), jnp.float32),
            pltpu.VMEM((chunk,), jnp.float32),
            pltpu.VMEM((chunk,), jnp.float32),
        ],
    )
    def k(t0_hbm, t1_hbm, t2_hbm, idx_hbm, out_hbm,
          v0, v1, v2, idx_v, o0, o1, o2):
        wid = lax.axis_index("s") * 2 + lax.axis_index("c")
        base = wid * per_w
        pltpu.sync_copy(t0_hbm, v0)
        pltpu.sync_copy(t1_hbm, v1)
        pltpu.sync_copy(t2_hbm, v2)
        pltpu.sync_copy(idx_hbm.at[pl.ds(base, per_w)], idx_v)

        def outer(cix, _):
            def inner(v, _):
                idx16 = idx_v[pl.ds(cix * chunk + v * 16, 16)]
                o0[pl.ds(v * 16, 16)] = plsc.load_gather(v0, [idx16])
                o1[pl.ds(v * 16, 16)] = plsc.load_gather(v1, [idx16])
                o2[pl.ds(v * 16, 16)] = plsc.load_gather(v2, [idx16])
                return ()

            lax.fori_loop(0, chunk // 16, inner, (), unroll=False)
            off = base + cix * chunk
            pltpu.sync_copy(o0, out_hbm.at[0, 0, pl.ds(off, chunk)])
            pltpu.sync_copy(o1, out_hbm.at[1, 0, pl.ds(off, chunk)])
            pltpu.sync_copy(o2, out_hbm.at[2, 0, pl.ds(off, chunk)])
            return ()

        lax.fori_loop(0, per_w // chunk, outer, (), unroll=False)

    return k(t0, t1, t2, idxf)


def _conv_body(nb_ref, xr_ref, w1a_ref, wb_ref, w2_ref, o_ref):
    xr = xr_ref[0]
    nb2 = nb_ref[:, 0, 0, 0, :]
    r = xr.shape[1]

    base = lax.dot_general(
        xr, wb_ref[...], (((0,), (0,)), ((), ())),
        preferred_element_type=jnp.float32)
    base2 = jnp.broadcast_to(base[:, None, :], (r, _K, 64)).reshape(r * _K, 64)
    h1 = _leaky(lax.dot_general(
        nb2, w1a_ref[...], (((0,), (0,)), ((), ())),
        preferred_element_type=jnp.float32) + base2)
    h2 = _leaky(lax.dot_general(
        h1, w2_ref[...], (((1,), (0,)), ((), ())),
        preferred_element_type=jnp.float32))
    o_ref[0] = jnp.max(h2.reshape(r, _K, 64), axis=1)


@jax.jit
def kernel(x, W1, W2):
    b, c, n = x.shape
    xp = jnp.pad(x, ((0, 0), (0, 8 - c), (0, 0)))

    idx = pl.pallas_call(
        _topk_body,
        grid=(b, n // _R),
        in_specs=[
            pl.BlockSpec((1, 8, n), lambda i, j: (i, 0, 0)),
            pl.BlockSpec((1, 8, _R), lambda i, j: (i, 0, j)),
        ],
        out_specs=pl.BlockSpec((1, _R, _K), lambda i, j: (i, j, 0)),
        out_shape=jax.ShapeDtypeStruct((b, n, _K), jnp.int32),
    )(xp, xp)

    planes = [x[:, k, :].reshape(b * n) for k in range(c)]
    gathered = _sc_gather(*planes, idx.reshape(b * n * _K))
    nbr = gathered.reshape(3, b, n // _R, 1, _R * _K)

    w1a = W1[:, :c].T
    wb = jnp.pad((W1[:, c:] - W1[:, :c]).T, ((0, 8 - c), (0, 0)))
    w2t = W2.T

    out = pl.pallas_call(
        _conv_body,
        grid=(b, n // _R),
        in_specs=[
            pl.BlockSpec((3, 1, 1, 1, _R * _K), lambda i, j: (0, i, j, 0, 0)),
            pl.BlockSpec((1, 8, _R), lambda i, j: (i, 0, j)),
            pl.BlockSpec((3, 64), lambda i, j: (0, 0)),
            pl.BlockSpec((8, 64), lambda i, j: (0, 0)),
            pl.BlockSpec((64, 64), lambda i, j: (0, 0)),
        ],
        out_specs=pl.BlockSpec((1, _R, 64), lambda i, j: (i, j, 0)),
        out_shape=jax.ShapeDtypeStruct((b, n, 64), jnp.float32),
    )(nbr, xp, w1a, wb, w2t)
    return jnp.swapaxes(out, 1, 2)
